# baseline (device time: 208991 ns/iter reference)
import jax
import jax.numpy as jnp
from jax import lax
from jax.experimental import pallas as pl
from jax.experimental.pallas import tpu as pltpu

N_DEV = 32

_PLANE = [(0, 0), (1, 0), (1, 1), (0, 1), (0, 2), (1, 2), (1, 3), (0, 3)]
_MESH_IDX = {
    (x, y, z): 8 * z + _PLANE.index((x, y))
    for z in range(4) for (x, y) in _PLANE
}
_P_YZ = [(0, 0), (1, 0), (2, 0), (3, 0), (3, 1), (2, 1), (1, 1), (0, 1),
         (0, 2), (1, 2), (2, 2), (3, 2), (3, 3), (2, 3), (1, 3), (0, 3)]
_RING = ([(0, y, z) for (y, z) in _P_YZ]
         + [(1, y, z) for (y, z) in reversed(_P_YZ)])
_PERM = [_MESH_IDX[c] for c in _RING]
_INV = [0] * N_DEV
for _p, _m in enumerate(_PERM):
    _INV[_m] = _p


def kernel(x, w_mat):
    m_per, k = x.shape
    _, n_per = w_mat.shape
    m_total = N_DEV * m_per

    perm = jnp.asarray(_PERM, dtype=jnp.int32)
    inv = jnp.asarray(_INV, dtype=jnp.int32)
    my_idx = lax.axis_index("i")
    rp = inv[my_idx]
    meta = jnp.concatenate([
        jnp.stack([perm[(rp + 1) % N_DEV], perm[(rp - 1) % N_DEV]]),
        perm[(rp - jnp.arange(1, N_DEV // 2 + 1)) % N_DEV],
        perm[(rp + jnp.arange(1, N_DEV // 2)) % N_DEV],
    ]).astype(jnp.int32)

    def body(x_ref, w_ref, meta_ref, out_ref, xg_ref, w_bf_ref, amax_ref,
             ring_send_sems, ring_recv_sems,
             left_send_sems, left_recv_sems,
             amax_send_sems, amax_recv_sems):
        my = lax.axis_index("i")
        right = meta_ref[0]
        left = meta_ref[1]

        barrier = pltpu.get_barrier_semaphore()
        for nbr in (left, right):
            pl.semaphore_signal(
                barrier, inc=1,
                device_id=(nbr,), device_id_type=pl.DeviceIdType.MESH,
            )

        xg_ref[0] = x_ref[...].astype(jnp.bfloat16)
        pl.semaphore_wait(barrier, 2)

        NQ = 4
        kq = k // NQ
        halves = tuple(pl.ds(j * kq, kq) for j in range(NQ))

        sends_r = {}
        sends_l = {}

        def start_r(h, j):
            r = pltpu.make_async_remote_copy(
                src_ref=xg_ref.at[h - 1, :, halves[j]],
                dst_ref=xg_ref.at[h, :, halves[j]],
                send_sem=ring_send_sems.at[h - 1, j],
                recv_sem=ring_recv_sems.at[h - 1, j],
                device_id=(right,),
                device_id_type=pl.DeviceIdType.MESH,
            )
            r.start()
            sends_r[(h, j)] = r

        def start_l(h, j):
            src = 0 if h == 1 else (N_DEV + 1 - h)
            r = pltpu.make_async_remote_copy(
                src_ref=xg_ref.at[src, :, halves[j]],
                dst_ref=xg_ref.at[N_DEV - h, :, halves[j]],
                send_sem=left_send_sems.at[h - 1, j],
                recv_sem=left_recv_sems.at[h - 1, j],
                device_id=(left,),
                device_id_type=pl.DeviceIdType.MESH,
            )
            r.start()
            sends_l[(h, j)] = r

        N_R = N_DEV // 2
        N_L = N_DEV // 2 - 1

        amax_acc = [None]

        def track(y):
            m = jnp.max(jnp.abs(y))
            amax_acc[0] = m if amax_acc[0] is None else jnp.maximum(
                amax_acc[0], m)

        for j in range(NQ):
            start_r(1, j)
            start_l(1, j)
        w_bf_ref[...] = w_ref[...].astype(jnp.bfloat16)
        y0 = jnp.dot(xg_ref[0], w_bf_ref[...],
                     preferred_element_type=jnp.float32)
        out_ref[pl.ds(my * m_per, m_per), :] = y0
        track(y0)

        def gemm_slot(slot, origin):
            y = jnp.dot(xg_ref[slot], w_bf_ref[...],
                        preferred_element_type=jnp.float32)
            out_ref[pl.ds(origin * m_per, m_per), :] = y
            track(y)

        def gemm_r(h):
            gemm_slot(h, meta_ref[2 + (h - 1)])

        def gemm_l(h):
            gemm_slot(N_DEV - h, meta_ref[2 + N_DEV // 2 + (h - 1)])

        for h in range(1, N_R):
            for j in range(NQ):
                sends_r[(h, j)].wait_recv()
                if h + 1 < N_R or j < 2:
                    start_r(h + 1, j)
                sends_l[(h, j)].wait_recv()
                if h < N_L:
                    start_l(h + 1, j)
                elif j >= 2:
                    start_l(N_R, j)

                if j == 1 and h >= 2:
                    gemm_l(h - 1)

            gemm_r(h)

        sends_r[(N_R, 0)].wait_recv()
        sends_r[(N_R, 1)].wait_recv()
        gemm_l(N_L)
        sends_l[(N_R, 2)].wait_recv()
        sends_l[(N_R, 3)].wait_recv()
        gemm_r(N_R)

        for r in sends_r.values():
            r.wait_send()
        for r in sends_l.values():
            r.wait_send()

        amax_run = amax_acc[0]

        amax_ref[0] = jnp.full((128,), amax_run, dtype=jnp.float32)
        sends = []
        for d in range(1, N_DEV):
            tgt = lax.rem(my + d, N_DEV)
            rdma = pltpu.make_async_remote_copy(
                src_ref=amax_ref.at[0],
                dst_ref=amax_ref.at[N_DEV - d],
                send_sem=amax_send_sems.at[d - 1],
                recv_sem=amax_recv_sems.at[d - 1],
                device_id=(tgt,),
                device_id_type=pl.DeviceIdType.MESH,
            )
            rdma.start()
            sends.append(rdma)
        for rdma in sends:
            rdma.wait_recv()
        for rdma in sends:
            rdma.wait_send()

        g_amax = jnp.max(amax_ref[...])
        scale = g_amax / 127.0
        inv_scale = 127.0 / g_amax
        y_all = out_ref[...]
        q = jnp.clip(jnp.round(y_all * inv_scale), -127.0, 127.0)
        out_ref[...] = q * scale

    return pl.pallas_call(
        body,
        out_shape=jax.ShapeDtypeStruct((m_total, n_per), jnp.float32),
        in_specs=[
            pl.BlockSpec(memory_space=pltpu.VMEM),
            pl.BlockSpec(memory_space=pltpu.VMEM),
            pl.BlockSpec(memory_space=pltpu.SMEM),
        ],
        out_specs=pl.BlockSpec(memory_space=pltpu.VMEM),
        scratch_shapes=[
            pltpu.VMEM((N_DEV, m_per, k), jnp.bfloat16),
            pltpu.VMEM((k, n_per), jnp.bfloat16),
            pltpu.VMEM((N_DEV, 128), jnp.float32),
            pltpu.SemaphoreType.DMA((N_DEV // 2, 4)),
            pltpu.SemaphoreType.DMA((N_DEV // 2, 4)),
            pltpu.SemaphoreType.DMA((N_DEV // 2, 4)),
            pltpu.SemaphoreType.DMA((N_DEV // 2, 4)),
            pltpu.SemaphoreType.DMA((N_DEV - 1,)),
            pltpu.SemaphoreType.DMA((N_DEV - 1,)),
        ],
        compiler_params=pltpu.CompilerParams(
            collective_id=0,
            vmem_limit_bytes=100 * 1024 * 1024,
        ),
    )(x, w_mat, meta)


# device time: 208205 ns/iter; 1.0038x vs baseline; 1.0038x over previous
import jax
import jax.numpy as jnp
from jax import lax
from jax.experimental import pallas as pl
from jax.experimental.pallas import tpu as pltpu

N_DEV = 32

_PLANE = [(0, 0), (1, 0), (1, 1), (0, 1), (0, 2), (1, 2), (1, 3), (0, 3)]
_MESH_IDX = {
    (x, y, z): 8 * z + _PLANE.index((x, y))
    for z in range(4) for (x, y) in _PLANE
}
_P_YZ = [(0, 0), (1, 0), (2, 0), (3, 0), (3, 1), (2, 1), (1, 1), (0, 1),
         (0, 2), (1, 2), (2, 2), (3, 2), (3, 3), (2, 3), (1, 3), (0, 3)]
_RING = ([(0, y, z) for (y, z) in _P_YZ]
         + [(1, y, z) for (y, z) in reversed(_P_YZ)])
_PERM = [_MESH_IDX[c] for c in _RING]
_INV = [0] * N_DEV
for _p, _m in enumerate(_PERM):
    _INV[_m] = _p


def kernel(x, w_mat):
    m_per, k = x.shape
    _, n_per = w_mat.shape
    m_total = N_DEV * m_per

    perm = jnp.asarray(_PERM, dtype=jnp.int32)
    inv = jnp.asarray(_INV, dtype=jnp.int32)
    my_idx = lax.axis_index("i")
    rp = inv[my_idx]
    meta = jnp.concatenate([
        jnp.stack([perm[(rp + 1) % N_DEV], perm[(rp - 1) % N_DEV]]),
        perm[(rp - jnp.arange(1, N_DEV // 2 + 1)) % N_DEV],
        perm[(rp + jnp.arange(1, N_DEV // 2)) % N_DEV],
    ]).astype(jnp.int32)

    def body(x_ref, w_ref, meta_ref, out_ref, xg_ref, w_bf_ref, amax_ref,
             ring_send_sems, ring_recv_sems,
             left_send_sems, left_recv_sems,
             amax_send_sems, amax_recv_sems):
        my = lax.axis_index("i")
        right = meta_ref[0]
        left = meta_ref[1]

        barrier = pltpu.get_barrier_semaphore()
        for nbr in (left, right):
            pl.semaphore_signal(
                barrier, inc=1,
                device_id=(nbr,), device_id_type=pl.DeviceIdType.MESH,
            )

        xg_ref[0] = x_ref[...].astype(jnp.bfloat16)
        pl.semaphore_wait(barrier, 2)

        kh = k // 2
        halves = (pl.ds(0, kh), pl.ds(kh, kh))

        sends_r = {}
        sends_l = {}

        def start_r(h, j):
            r = pltpu.make_async_remote_copy(
                src_ref=xg_ref.at[h - 1, :, halves[j]],
                dst_ref=xg_ref.at[h, :, halves[j]],
                send_sem=ring_send_sems.at[h - 1, j],
                recv_sem=ring_recv_sems.at[h - 1, j],
                device_id=(right,),
                device_id_type=pl.DeviceIdType.MESH,
            )
            r.start()
            sends_r[(h, j)] = r

        def start_l(h, j):
            src = 0 if h == 1 else (N_DEV + 1 - h)
            r = pltpu.make_async_remote_copy(
                src_ref=xg_ref.at[src, :, halves[j]],
                dst_ref=xg_ref.at[N_DEV - h, :, halves[j]],
                send_sem=left_send_sems.at[h - 1, j],
                recv_sem=left_recv_sems.at[h - 1, j],
                device_id=(left,),
                device_id_type=pl.DeviceIdType.MESH,
            )
            r.start()
            sends_l[(h, j)] = r

        N_R = N_DEV // 2
        N_L = N_DEV // 2 - 1

        amax_acc = [None]

        def track(y):
            m = jnp.max(jnp.abs(y))
            amax_acc[0] = m if amax_acc[0] is None else jnp.maximum(
                amax_acc[0], m)

        start_r(1, 0)
        start_r(1, 1)
        start_l(1, 0)
        start_l(1, 1)
        w_bf_ref[...] = w_ref[...].astype(jnp.bfloat16)
        y0 = jnp.dot(xg_ref[0], w_bf_ref[...],
                     preferred_element_type=jnp.float32)
        out_ref[pl.ds(my * m_per, m_per), :] = y0
        track(y0)

        def gemm_slot(slot, origin):
            y = jnp.dot(xg_ref[slot], w_bf_ref[...],
                        preferred_element_type=jnp.float32)
            out_ref[pl.ds(origin * m_per, m_per), :] = y
            track(y)

        def gemm_r(h):
            gemm_slot(h, meta_ref[2 + (h - 1)])

        def gemm_l(h):
            gemm_slot(N_DEV - h, meta_ref[2 + N_DEV // 2 + (h - 1)])

        for h in range(1, N_R):
            sends_r[(h, 0)].wait_recv()
            start_r(h + 1, 0)
            sends_l[(h, 0)].wait_recv()
            if h < N_L:
                start_l(h + 1, 0)

            if h >= 2:
                gemm_l(h - 1)

            sends_r[(h, 1)].wait_recv()
            if h + 1 < N_R:
                start_r(h + 1, 1)
            sends_l[(h, 1)].wait_recv()
            if h < N_L:
                start_l(h + 1, 1)
            if h == N_L:
                start_l(N_R, 1)

            gemm_r(h)

        sends_r[(N_R, 0)].wait_recv()
        gemm_l(N_L)
        sends_l[(N_R, 1)].wait_recv()
        gemm_r(N_R)

        for r in sends_r.values():
            r.wait_send()
        for r in sends_l.values():
            r.wait_send()

        amax_run = amax_acc[0]

        amax_ref[0] = jnp.full((128,), amax_run, dtype=jnp.float32)
        sends = []
        for d in range(1, N_DEV):
            tgt = lax.rem(my + d, N_DEV)
            rdma = pltpu.make_async_remote_copy(
                src_ref=amax_ref.at[0],
                dst_ref=amax_ref.at[N_DEV - d],
                send_sem=amax_send_sems.at[d - 1],
                recv_sem=amax_recv_sems.at[d - 1],
                device_id=(tgt,),
                device_id_type=pl.DeviceIdType.MESH,
            )
            rdma.start()
            sends.append(rdma)
        for rdma in sends:
            rdma.wait_recv()
        for rdma in sends:
            rdma.wait_send()

        g_amax = jnp.max(amax_ref[...])
        scale = g_amax / 127.0
        inv_scale = 127.0 / g_amax
        y_all = out_ref[...]
        q = jnp.clip(jnp.round(y_all * inv_scale), -127.0, 127.0)
        out_ref[...] = q * scale

    return pl.pallas_call(
        body,
        out_shape=jax.ShapeDtypeStruct((m_total, n_per), jnp.float32),
        in_specs=[
            pl.BlockSpec(memory_space=pltpu.VMEM),
            pl.BlockSpec(memory_space=pltpu.VMEM),
            pl.BlockSpec(memory_space=pltpu.SMEM),
        ],
        out_specs=pl.BlockSpec(memory_space=pltpu.VMEM),
        scratch_shapes=[
            pltpu.VMEM((N_DEV, m_per, k), jnp.bfloat16),
            pltpu.VMEM((k, n_per), jnp.bfloat16),
            pltpu.VMEM((N_DEV, 128), jnp.float32),
            pltpu.SemaphoreType.DMA((N_DEV // 2, 2)),
            pltpu.SemaphoreType.DMA((N_DEV // 2, 2)),
            pltpu.SemaphoreType.DMA((N_DEV // 2, 2)),
            pltpu.SemaphoreType.DMA((N_DEV // 2, 2)),
            pltpu.SemaphoreType.DMA((N_DEV - 1,)),
            pltpu.SemaphoreType.DMA((N_DEV - 1,)),
        ],
        compiler_params=pltpu.CompilerParams(
            collective_id=0,
            vmem_limit_bytes=100 * 1024 * 1024,
        ),
    )(x, w_mat, meta)
